# baseline (speedup 1.0000x reference)
import functools

import jax
import jax.numpy as jnp
from jax import lax
from jax.experimental import pallas as pl
from jax.experimental.pallas import tpu as pltpu
from jax.experimental.pallas import tpu_sc as plsc

N = 10000
E = 320000
F = 128
H = 128
C = 16

NC = 2
NS = 16
NW = NC * NS
L = 16

EPT = E // NW
NOFF = 10240
NP = 2 * NOFF
RED = NP // NS


def _counts_body(src_ref, dst_ref, out_ref, pk_ref, cnt_ref, sbuf, dbuf, pbuf,
                 shared, red, acc):
    cid = lax.axis_index("c")
    sid = lax.axis_index("s")
    wid = sid * NC + cid
    zeros = jnp.zeros((L,), jnp.float32)
    ones = jnp.ones((L,), jnp.float32)

    def zero_body(i, _):
        cnt_ref[pl.ds(i * L, L)] = zeros
        return 0

    lax.fori_loop(0, NP // L, zero_body, 0)

    eoff = wid * EPT
    pltpu.sync_copy(src_ref.at[pl.ds(eoff, EPT)], sbuf)
    pltpu.sync_copy(dst_ref.at[pl.ds(eoff, EPT)], dbuf)

    @plsc.parallel_loop(0, EPT // L, 1, unroll=4)
    def count_body(i):
        s16 = sbuf[pl.ds(i * L, L)]
        d16 = dbuf[pl.ds(i * L, L)]
        plsc.addupdate_scatter(cnt_ref, [s16], ones)
        plsc.addupdate_scatter(cnt_ref, [d16 + NOFF], ones)
        pbuf[pl.ds(i * L, L)] = s16 | (d16 << 16)
    pltpu.sync_copy(pbuf, pk_ref.at[pl.ds(eoff, EPT)])

    pltpu.sync_copy(cnt_ref, shared.at[sid])
    plsc.subcore_barrier()

    col0 = sid * RED

    def zero_acc(i, _):
        acc[pl.ds(i * L, L)] = zeros
        return 0

    lax.fori_loop(0, RED // L, zero_acc, 0)

    for t in range(NS):
        pltpu.sync_copy(shared.at[t, pl.ds(col0, RED)], red)

        def add_body(i, _):
            acc[pl.ds(i * L, L)] = acc[pl.ds(i * L, L)] + red[pl.ds(i * L, L)]
            return 0

        lax.fori_loop(0, RED // L, add_body, 0)

    pltpu.sync_copy(acc, out_ref.at[cid, pl.ds(col0, RED)])


_sc_counts = pl.kernel(
    _counts_body,
    out_type=(jax.ShapeDtypeStruct((NC, NP), jnp.float32),
              jax.ShapeDtypeStruct((E,), jnp.int32)),
    mesh=plsc.VectorSubcoreMesh(core_axis_name="c", subcore_axis_name="s",
                                num_cores=NC, num_subcores=NS),
    scratch_types=[
        pltpu.VMEM((NP,), jnp.float32),
        pltpu.VMEM((EPT,), jnp.int32),
        pltpu.VMEM((EPT,), jnp.int32),
        pltpu.VMEM((EPT,), jnp.int32),
        pltpu.VMEM_SHARED((NS, NP), jnp.float32),
        pltpu.VMEM((RED,), jnp.float32),
        pltpu.VMEM((RED,), jnp.float32),
    ],
    compiler_params=pltpu.CompilerParams(needs_layout_passes=False),
)

FPT = F // NW
CHB = 20000
NPAIR = E // (2 * CHB)


def _agg_body(p_ref, pk_ref, out_hbm,
              in0, in1, ou0, ou1, ou2, ou3,
              pb0, pb1, sem_0, sem_1):
    cid = lax.axis_index("c")
    sid = lax.axis_index("s")
    wid = sid * NC + cid
    v0 = wid * (FPT * N)
    zeros = jnp.zeros((L,), jnp.float32)
    ins = (in0, in1)
    outs = (ou0, ou1, ou2, ou3)

    def edge_dma(e0, pbuf, sem):
        return pltpu.make_async_copy(pk_ref.at[pl.ds(e0, CHB)], pbuf, sem)

    edge_dma(0, pb0, sem_0).start()

    pv0 = wid * (2 * N)
    for c in range(2):
        pltpu.sync_copy(p_ref.at[pl.ds(pv0 + c * N, N)], ins[c])

    def zero_body(i, _):
        for c in range(FPT):
            outs[c][pl.ds(i * L, L)] = zeros
        return 0

    lax.fori_loop(0, N // L, zero_body, 0)

    himask = jnp.full((L,), -65536, jnp.int32)

    def process(pbuf):
        @plsc.parallel_loop(0, CHB // L, 1, unroll=8)
        def _(i):
            p16 = pbuf[pl.ds(i * L, L)]
            s16 = p16 & 0xFFFF
            d16 = lax.shift_right_logical(p16, 16)
            for c in range(2):
                g = plsc.load_gather(ins[c], [s16])
                fa = plsc.bitcast(lax.shift_left(g, 16), jnp.float32)
                fb = plsc.bitcast(g & himask, jnp.float32)
                plsc.addupdate_scatter(outs[2 * c], [d16], fa)
                plsc.addupdate_scatter(outs[2 * c + 1], [d16], fb)

    def pair_body(j, _):
        e0 = 2 * j * CHB
        edge_dma(e0 + CHB, pb1, sem_1).start()
        edge_dma(e0, pb0, sem_0).wait()
        process(pb0)

        @pl.when(j + 1 < NPAIR)
        def _():
            edge_dma(e0 + 2 * CHB, pb0, sem_0).start()

        edge_dma(e0 + CHB, pb1, sem_1).wait()
        process(pb1)
        return 0

    lax.fori_loop(0, NPAIR, pair_body, 0)

    for c in range(FPT):
        pltpu.sync_copy(outs[c], out_hbm.at[pl.ds(v0 + c * N, N)])


_sc_aggregate = pl.kernel(
    _agg_body,
    out_type=jax.ShapeDtypeStruct((F * N,), jnp.float32),
    mesh=plsc.VectorSubcoreMesh(core_axis_name="c", subcore_axis_name="s",
                                num_cores=NC, num_subcores=NS),
    scratch_types=[
        pltpu.VMEM((N,), jnp.int32),
        pltpu.VMEM((N,), jnp.int32),
        pltpu.VMEM((N,), jnp.float32),
        pltpu.VMEM((N,), jnp.float32),
        pltpu.VMEM((N,), jnp.float32),
        pltpu.VMEM((N,), jnp.float32),
        pltpu.VMEM((CHB,), jnp.int32),
        pltpu.VMEM((CHB,), jnp.int32),
        pltpu.SemaphoreType.DMA,
        pltpu.SemaphoreType.DMA,
    ],
    compiler_params=pltpu.CompilerParams(needs_layout_passes=False),
)


def _pack_pairs(even, odd):
    lo = lax.bitcast_convert_type(even.astype(jnp.bfloat16), jnp.uint16)
    hi = lax.bitcast_convert_type(odd.astype(jnp.bfloat16), jnp.uint16)
    w = lo.astype(jnp.uint32) | (hi.astype(jnp.uint32) << 16)
    return lax.bitcast_convert_type(w, jnp.int32)


def _prep_body(x_ref, cs_ref, cd_ref, p_ref, ns_ref, nd_ref):
    cs = cs_ref[...]
    cd = cd_ref[...]
    ns = lax.rsqrt(jnp.maximum(cs[0:1] + cs[1:2], 1.0))
    nd = lax.rsqrt(jnp.maximum(cd[0:1] + cd[1:2], 1.0))
    ns_ref[...] = ns
    nd_ref[...] = nd
    r = lax.broadcasted_iota(jnp.int32, (F // 2, F), 0)
    c = lax.broadcasted_iota(jnp.int32, (F // 2, F), 1)
    sel_e = (c == 2 * r).astype(jnp.float32)
    sel_o = (c == 2 * r + 1).astype(jnp.float32)
    xb = x_ref[...]
    xte = lax.dot_general(sel_e, xb, (((1,), (1,)), ((), ())),
                          preferred_element_type=jnp.float32) * ns
    xto = lax.dot_general(sel_o, xb, (((1,), (1,)), ((), ())),
                          preferred_element_type=jnp.float32) * ns
    p_ref[...] = _pack_pairs(xte, xto)


_tc_prep = pl.pallas_call(
    _prep_body,
    out_shape=[
        jax.ShapeDtypeStruct((F // 2, N), jnp.int32),
        jax.ShapeDtypeStruct((1, N), jnp.float32),
        jax.ShapeDtypeStruct((1, N), jnp.float32),
    ],
)


def _mid_body(s_ref, we_ref, wo_ref, be_ref, bo_ref, nd_ref, ns_ref, h_ref):
    sb = s_ref[...] * nd_ref[...]
    ns = ns_ref[...]
    he = lax.dot_general(we_ref[...], sb, (((0,), (0,)), ((), ())),
                         preferred_element_type=jnp.float32) + be_ref[...]
    ho = lax.dot_general(wo_ref[...], sb, (((0,), (0,)), ((), ())),
                         preferred_element_type=jnp.float32) + bo_ref[...]
    he = jnp.maximum(he, 0.0) * ns
    ho = jnp.maximum(ho, 0.0) * ns
    h_ref[...] = _pack_pairs(he, ho)


_tc_mid = pl.pallas_call(
    _mid_body,
    out_shape=jax.ShapeDtypeStruct((H // 2, N), jnp.int32),
)


def _head_body(s_ref, w2_ref, b2_ref, wfc_ref, bfc_ref, nd_ref, o_ref):
    sb = s_ref[...] * nd_ref[...]
    h2 = lax.dot_general(sb, w2_ref[...], (((0,), (0,)), ((), ())),
                         preferred_element_type=jnp.float32)
    h2 = h2 + b2_ref[...]
    o_ref[...] = lax.dot_general(h2, wfc_ref[...], (((1,), (0,)), ((), ())),
                                 preferred_element_type=jnp.float32) + bfc_ref[...]


_tc_head = pl.pallas_call(
    _head_body,
    out_shape=jax.ShapeDtypeStruct((N, C), jnp.float32),
)


def kernel(x, edge_index, W1, b1, W2, b2, Wfc, bfc):
    ei = edge_index.astype(jnp.int32)
    src = ei[0]
    dst = ei[1]
    counts, pk = _sc_counts(src, dst)
    cs = counts[:, :N]
    cd = counts[:, NOFF:NOFF + N]
    p1t, ns, nd = _tc_prep(x, cs, cd)
    s1t = _sc_aggregate(p1t.reshape(-1), pk).reshape(F, N)
    h1t = _tc_mid(s1t, W1[:, 0::2], W1[:, 1::2],
                  b1[0::2].reshape(H // 2, 1), b1[1::2].reshape(H // 2, 1),
                  nd, ns)
    s2t = _sc_aggregate(h1t.reshape(-1), pk).reshape(F, N)
    out = _tc_head(s2t, W2, b2.reshape(1, H), Wfc, bfc.reshape(1, C), nd)
    return out


# speedup vs baseline: 1.0124x; 1.0124x over previous
import functools

import jax
import jax.numpy as jnp
from jax import lax
from jax.experimental import pallas as pl
from jax.experimental.pallas import tpu as pltpu
from jax.experimental.pallas import tpu_sc as plsc

N = 10000
E = 320000
F = 128
H = 128
C = 16

NC = 2
NS = 16
NW = NC * NS
L = 16

EPT = E // NW
NOFF = 10240
NP = 2 * NOFF
RED = NP // NS


def _counts_body(src_ref, dst_ref, out_ref, pk_ref, cnt_ref, sbuf, dbuf, pbuf,
                 shared, red, acc):
    cid = lax.axis_index("c")
    sid = lax.axis_index("s")
    wid = sid * NC + cid
    zeros = jnp.zeros((L,), jnp.float32)
    ones = jnp.ones((L,), jnp.float32)

    def zero_body(i, _):
        cnt_ref[pl.ds(i * L, L)] = zeros
        return 0

    lax.fori_loop(0, NP // L, zero_body, 0)

    eoff = wid * EPT
    pltpu.sync_copy(src_ref.at[pl.ds(eoff, EPT)], sbuf)
    pltpu.sync_copy(dst_ref.at[pl.ds(eoff, EPT)], dbuf)

    @plsc.parallel_loop(0, EPT // L, 1, unroll=4)
    def count_body(i):
        s16 = sbuf[pl.ds(i * L, L)]
        d16 = dbuf[pl.ds(i * L, L)]
        plsc.addupdate_scatter(cnt_ref, [s16], ones)
        plsc.addupdate_scatter(cnt_ref, [d16 + NOFF], ones)
        pbuf[pl.ds(i * L, L)] = s16 | (d16 << 16)
    pltpu.sync_copy(pbuf, pk_ref.at[pl.ds(eoff, EPT)])

    pltpu.sync_copy(cnt_ref, shared.at[sid])
    plsc.subcore_barrier()

    col0 = sid * RED

    def zero_acc(i, _):
        acc[pl.ds(i * L, L)] = zeros
        return 0

    lax.fori_loop(0, RED // L, zero_acc, 0)

    for t in range(NS):
        pltpu.sync_copy(shared.at[t, pl.ds(col0, RED)], red)

        def add_body(i, _):
            acc[pl.ds(i * L, L)] = acc[pl.ds(i * L, L)] + red[pl.ds(i * L, L)]
            return 0

        lax.fori_loop(0, RED // L, add_body, 0)

    pltpu.sync_copy(acc, out_ref.at[cid, pl.ds(col0, RED)])


_sc_counts = pl.kernel(
    _counts_body,
    out_type=(jax.ShapeDtypeStruct((NC, NP), jnp.float32),
              jax.ShapeDtypeStruct((E,), jnp.int32)),
    mesh=plsc.VectorSubcoreMesh(core_axis_name="c", subcore_axis_name="s",
                                num_cores=NC, num_subcores=NS),
    scratch_types=[
        pltpu.VMEM((NP,), jnp.float32),
        pltpu.VMEM((EPT,), jnp.int32),
        pltpu.VMEM((EPT,), jnp.int32),
        pltpu.VMEM((EPT,), jnp.int32),
        pltpu.VMEM_SHARED((NS, NP), jnp.float32),
        pltpu.VMEM((RED,), jnp.float32),
        pltpu.VMEM((RED,), jnp.float32),
    ],
    compiler_params=pltpu.CompilerParams(needs_layout_passes=False),
)

FPT = F // NW
CHB = 20000
NPAIR = E // (2 * CHB)


def _agg_body(p_ref, pk_ref, out_hbm,
              in0, in1, ou0, ou1, ou2, ou3,
              pb0, pb1, sem_0, sem_1):
    cid = lax.axis_index("c")
    sid = lax.axis_index("s")
    wid = sid * NC + cid
    v0 = wid * (FPT * N)
    zeros = jnp.zeros((L,), jnp.float32)
    ins = (in0, in1)
    outs = (ou0, ou1, ou2, ou3)

    def edge_dma(e0, pbuf, sem):
        return pltpu.make_async_copy(pk_ref.at[pl.ds(e0, CHB)], pbuf, sem)

    edge_dma(0, pb0, sem_0).start()

    pv0 = wid * (2 * N)
    for c in range(2):
        pltpu.sync_copy(p_ref.at[pl.ds(pv0 + c * N, N)], ins[c])

    def zero_body(i, _):
        for c in range(FPT):
            outs[c][pl.ds(i * L, L)] = zeros
        return 0

    lax.fori_loop(0, N // L, zero_body, 0)

    himask = jnp.full((L,), -65536, jnp.int32)

    def process(pbuf):
        @plsc.parallel_loop(0, CHB // L, 1, unroll=4)
        def _(i):
            p16 = pbuf[pl.ds(i * L, L)]
            s16 = p16 & 0xFFFF
            d16 = lax.shift_right_logical(p16, 16)
            for c in range(2):
                g = plsc.load_gather(ins[c], [s16])
                fa = plsc.bitcast(lax.shift_left(g, 16), jnp.float32)
                fb = plsc.bitcast(g & himask, jnp.float32)
                plsc.addupdate_scatter(outs[2 * c], [d16], fa)
                plsc.addupdate_scatter(outs[2 * c + 1], [d16], fb)

    def pair_body(j, _):
        e0 = 2 * j * CHB
        edge_dma(e0 + CHB, pb1, sem_1).start()
        edge_dma(e0, pb0, sem_0).wait()
        process(pb0)

        @pl.when(j + 1 < NPAIR)
        def _():
            edge_dma(e0 + 2 * CHB, pb0, sem_0).start()

        edge_dma(e0 + CHB, pb1, sem_1).wait()
        process(pb1)
        return 0

    lax.fori_loop(0, NPAIR, pair_body, 0)

    for c in range(FPT):
        pltpu.sync_copy(outs[c], out_hbm.at[pl.ds(v0 + c * N, N)])


_sc_aggregate = pl.kernel(
    _agg_body,
    out_type=jax.ShapeDtypeStruct((F * N,), jnp.float32),
    mesh=plsc.VectorSubcoreMesh(core_axis_name="c", subcore_axis_name="s",
                                num_cores=NC, num_subcores=NS),
    scratch_types=[
        pltpu.VMEM((N,), jnp.int32),
        pltpu.VMEM((N,), jnp.int32),
        pltpu.VMEM((N,), jnp.float32),
        pltpu.VMEM((N,), jnp.float32),
        pltpu.VMEM((N,), jnp.float32),
        pltpu.VMEM((N,), jnp.float32),
        pltpu.VMEM((CHB,), jnp.int32),
        pltpu.VMEM((CHB,), jnp.int32),
        pltpu.SemaphoreType.DMA,
        pltpu.SemaphoreType.DMA,
    ],
    compiler_params=pltpu.CompilerParams(needs_layout_passes=False),
)


def _pack_pairs(even, odd):
    lo = lax.bitcast_convert_type(even.astype(jnp.bfloat16), jnp.uint16)
    hi = lax.bitcast_convert_type(odd.astype(jnp.bfloat16), jnp.uint16)
    w = lo.astype(jnp.uint32) | (hi.astype(jnp.uint32) << 16)
    return lax.bitcast_convert_type(w, jnp.int32)


def _prep_body(x_ref, cs_ref, cd_ref, p_ref, ns_ref, nd_ref):
    cs = cs_ref[...]
    cd = cd_ref[...]
    ns = lax.rsqrt(jnp.maximum(cs[0:1] + cs[1:2], 1.0))
    nd = lax.rsqrt(jnp.maximum(cd[0:1] + cd[1:2], 1.0))
    ns_ref[...] = ns
    nd_ref[...] = nd
    r = lax.broadcasted_iota(jnp.int32, (F // 2, F), 0)
    c = lax.broadcasted_iota(jnp.int32, (F // 2, F), 1)
    sel_e = (c == 2 * r).astype(jnp.float32)
    sel_o = (c == 2 * r + 1).astype(jnp.float32)
    xb = x_ref[...]
    xte = lax.dot_general(sel_e, xb, (((1,), (1,)), ((), ())),
                          preferred_element_type=jnp.float32) * ns
    xto = lax.dot_general(sel_o, xb, (((1,), (1,)), ((), ())),
                          preferred_element_type=jnp.float32) * ns
    p_ref[...] = _pack_pairs(xte, xto)


_tc_prep = pl.pallas_call(
    _prep_body,
    out_shape=[
        jax.ShapeDtypeStruct((F // 2, N), jnp.int32),
        jax.ShapeDtypeStruct((1, N), jnp.float32),
        jax.ShapeDtypeStruct((1, N), jnp.float32),
    ],
)


def _mid_body(s_ref, we_ref, wo_ref, be_ref, bo_ref, nd_ref, ns_ref, h_ref):
    sb = s_ref[...] * nd_ref[...]
    ns = ns_ref[...]
    he = lax.dot_general(we_ref[...], sb, (((0,), (0,)), ((), ())),
                         preferred_element_type=jnp.float32) + be_ref[...]
    ho = lax.dot_general(wo_ref[...], sb, (((0,), (0,)), ((), ())),
                         preferred_element_type=jnp.float32) + bo_ref[...]
    he = jnp.maximum(he, 0.0) * ns
    ho = jnp.maximum(ho, 0.0) * ns
    h_ref[...] = _pack_pairs(he, ho)


_tc_mid = pl.pallas_call(
    _mid_body,
    out_shape=jax.ShapeDtypeStruct((H // 2, N), jnp.int32),
)


def _head_body(s_ref, w2_ref, b2_ref, wfc_ref, bfc_ref, nd_ref, o_ref):
    sb = s_ref[...] * nd_ref[...]
    h2 = lax.dot_general(sb, w2_ref[...], (((0,), (0,)), ((), ())),
                         preferred_element_type=jnp.float32)
    h2 = h2 + b2_ref[...]
    o_ref[...] = lax.dot_general(h2, wfc_ref[...], (((1,), (0,)), ((), ())),
                                 preferred_element_type=jnp.float32) + bfc_ref[...]


_tc_head = pl.pallas_call(
    _head_body,
    out_shape=jax.ShapeDtypeStruct((N, C), jnp.float32),
)


def kernel(x, edge_index, W1, b1, W2, b2, Wfc, bfc):
    ei = edge_index.astype(jnp.int32)
    src = ei[0]
    dst = ei[1]
    counts, pk = _sc_counts(src, dst)
    cs = counts[:, :N]
    cd = counts[:, NOFF:NOFF + N]
    p1t, ns, nd = _tc_prep(x, cs, cd)
    s1t = _sc_aggregate(p1t.reshape(-1), pk).reshape(F, N)
    h1t = _tc_mid(s1t, W1[:, 0::2], W1[:, 1::2],
                  b1[0::2].reshape(H // 2, 1), b1[1::2].reshape(H // 2, 1),
                  nd, ns)
    s2t = _sc_aggregate(h1t.reshape(-1), pk).reshape(F, N)
    out = _tc_head(s2t, W2, b2.reshape(1, H), Wfc, bfc.reshape(1, C), nd)
    return out


# speedup vs baseline: 1.0126x; 1.0002x over previous
import jax
import jax.numpy as jnp
from jax import lax
from jax.experimental import pallas as pl
from jax.experimental.pallas import tpu as pltpu
from jax.experimental.pallas import tpu_sc as plsc

N = 10000
E = 320000
F = 128
H = 128
C = 16

NC = 2
NS = 16
NW = NC * NS
L = 16

EPT = E // NW
NOFF = 10240
NP = 2 * NOFF
RED = NP // NS


def _counts_body(src_ref, dst_ref, out_ref, pk_ref, cnt_ref, sbuf, dbuf, pbuf,
                 shared, red, acc):
    cid = lax.axis_index("c")
    sid = lax.axis_index("s")
    wid = sid * NC + cid
    zeros = jnp.zeros((L,), jnp.float32)
    ones = jnp.ones((L,), jnp.float32)

    def zero_body(i, _):
        cnt_ref[pl.ds(i * L, L)] = zeros
        return 0

    lax.fori_loop(0, NP // L, zero_body, 0)

    eoff = wid * EPT
    pltpu.sync_copy(src_ref.at[pl.ds(eoff, EPT)], sbuf)
    pltpu.sync_copy(dst_ref.at[pl.ds(eoff, EPT)], dbuf)

    @plsc.parallel_loop(0, EPT // L, 1, unroll=4)
    def count_body(i):
        s16 = sbuf[pl.ds(i * L, L)]
        d16 = dbuf[pl.ds(i * L, L)]
        plsc.addupdate_scatter(cnt_ref, [s16], ones)
        plsc.addupdate_scatter(cnt_ref, [d16 + NOFF], ones)
        pbuf[pl.ds(i * L, L)] = s16 | (d16 << 16)
    pltpu.sync_copy(pbuf, pk_ref.at[pl.ds(eoff, EPT)])

    pltpu.sync_copy(cnt_ref, shared.at[sid])
    plsc.subcore_barrier()

    col0 = sid * RED

    def zero_acc(i, _):
        acc[pl.ds(i * L, L)] = zeros
        return 0

    lax.fori_loop(0, RED // L, zero_acc, 0)

    for t in range(NS):
        pltpu.sync_copy(shared.at[t, pl.ds(col0, RED)], red)

        def add_body(i, _):
            acc[pl.ds(i * L, L)] = acc[pl.ds(i * L, L)] + red[pl.ds(i * L, L)]
            return 0

        lax.fori_loop(0, RED // L, add_body, 0)

    pltpu.sync_copy(acc, out_ref.at[cid, pl.ds(col0, RED)])


_sc_counts = pl.kernel(
    _counts_body,
    out_type=(jax.ShapeDtypeStruct((NC, NP), jnp.float32),
              jax.ShapeDtypeStruct((E,), jnp.int32)),
    mesh=plsc.VectorSubcoreMesh(core_axis_name="c", subcore_axis_name="s",
                                num_cores=NC, num_subcores=NS),
    scratch_types=[
        pltpu.VMEM((NP,), jnp.float32),
        pltpu.VMEM((EPT,), jnp.int32),
        pltpu.VMEM((EPT,), jnp.int32),
        pltpu.VMEM((EPT,), jnp.int32),
        pltpu.VMEM_SHARED((NS, NP), jnp.float32),
        pltpu.VMEM((RED,), jnp.float32),
        pltpu.VMEM((RED,), jnp.float32),
    ],
    compiler_params=pltpu.CompilerParams(needs_layout_passes=False),
)

FPT = F // NW
CHB = 20000
NPAIR = E // (2 * CHB)


def _agg_body(p_ref, pk_ref, out_hbm,
              in0, in1, ou0, ou1, ou2, ou3,
              pb0, pb1, sem_0, sem_1):
    cid = lax.axis_index("c")
    sid = lax.axis_index("s")
    wid = sid * NC + cid
    v0 = wid * (FPT * N)
    zeros = jnp.zeros((L,), jnp.float32)
    ins = (in0, in1)
    outs = (ou0, ou1, ou2, ou3)

    def edge_dma(e0, pbuf, sem):
        return pltpu.make_async_copy(pk_ref.at[pl.ds(e0, CHB)], pbuf, sem)

    edge_dma(0, pb0, sem_0).start()

    pv0 = wid * (2 * N)
    for c in range(2):
        pltpu.sync_copy(p_ref.at[pl.ds(pv0 + c * N, N)], ins[c])

    def zero_body(i, _):
        for c in range(FPT):
            outs[c][pl.ds(i * L, L)] = zeros
        return 0

    lax.fori_loop(0, N // L, zero_body, 0)

    himask = jnp.full((L,), -65536, jnp.int32)

    def process(pbuf):
        @plsc.parallel_loop(0, CHB // L, 1, unroll=4)
        def _(i):
            p16 = pbuf[pl.ds(i * L, L)]
            s16 = p16 & 0xFFFF
            d16 = lax.shift_right_logical(p16, 16)
            for c in range(2):
                g = plsc.load_gather(ins[c], [s16])
                fa = plsc.bitcast(lax.shift_left(g, 16), jnp.float32)
                fb = plsc.bitcast(g & himask, jnp.float32)
                plsc.addupdate_scatter(outs[2 * c], [d16], fa)
                plsc.addupdate_scatter(outs[2 * c + 1], [d16], fb)

    def pair_body(j, _):
        e0 = 2 * j * CHB
        edge_dma(e0 + CHB, pb1, sem_1).start()
        edge_dma(e0, pb0, sem_0).wait()
        process(pb0)

        @pl.when(j + 1 < NPAIR)
        def _():
            edge_dma(e0 + 2 * CHB, pb0, sem_0).start()

        edge_dma(e0 + CHB, pb1, sem_1).wait()
        process(pb1)
        return 0

    lax.fori_loop(0, NPAIR, pair_body, 0)

    for c in range(FPT):
        pltpu.sync_copy(outs[c], out_hbm.at[pl.ds(v0 + c * N, N)])


_sc_aggregate = pl.kernel(
    _agg_body,
    out_type=jax.ShapeDtypeStruct((F * N,), jnp.float32),
    mesh=plsc.VectorSubcoreMesh(core_axis_name="c", subcore_axis_name="s",
                                num_cores=NC, num_subcores=NS),
    scratch_types=[
        pltpu.VMEM((N,), jnp.int32),
        pltpu.VMEM((N,), jnp.int32),
        pltpu.VMEM((N,), jnp.float32),
        pltpu.VMEM((N,), jnp.float32),
        pltpu.VMEM((N,), jnp.float32),
        pltpu.VMEM((N,), jnp.float32),
        pltpu.VMEM((CHB,), jnp.int32),
        pltpu.VMEM((CHB,), jnp.int32),
        pltpu.SemaphoreType.DMA,
        pltpu.SemaphoreType.DMA,
    ],
    compiler_params=pltpu.CompilerParams(needs_layout_passes=False),
)


def _pack_pairs(even, odd):
    lo = lax.bitcast_convert_type(even.astype(jnp.bfloat16), jnp.uint16)
    hi = lax.bitcast_convert_type(odd.astype(jnp.bfloat16), jnp.uint16)
    w = lo.astype(jnp.uint32) | (hi.astype(jnp.uint32) << 16)
    return lax.bitcast_convert_type(w, jnp.int32)


def _prep_body(x_ref, cs_ref, cd_ref, p_ref, ns_ref, nd_ref):
    cs = cs_ref[...]
    cd = cd_ref[...]
    ns = lax.rsqrt(jnp.maximum(cs[0:1] + cs[1:2], 1.0))
    nd = lax.rsqrt(jnp.maximum(cd[0:1] + cd[1:2], 1.0))
    ns_ref[...] = ns
    nd_ref[...] = nd
    r = lax.broadcasted_iota(jnp.int32, (F // 2, F), 0)
    c = lax.broadcasted_iota(jnp.int32, (F // 2, F), 1)
    sel_e = (c == 2 * r).astype(jnp.float32)
    sel_o = (c == 2 * r + 1).astype(jnp.float32)
    xb = x_ref[...]
    xte = lax.dot_general(sel_e, xb, (((1,), (1,)), ((), ())),
                          preferred_element_type=jnp.float32) * ns
    xto = lax.dot_general(sel_o, xb, (((1,), (1,)), ((), ())),
                          preferred_element_type=jnp.float32) * ns
    p_ref[...] = _pack_pairs(xte, xto)


_tc_prep = pl.pallas_call(
    _prep_body,
    out_shape=[
        jax.ShapeDtypeStruct((F // 2, N), jnp.int32),
        jax.ShapeDtypeStruct((1, N), jnp.float32),
        jax.ShapeDtypeStruct((1, N), jnp.float32),
    ],
)


def _mid_body(s_ref, we_ref, wo_ref, be_ref, bo_ref, nd_ref, ns_ref, h_ref):
    sb = s_ref[...] * nd_ref[...]
    ns = ns_ref[...]
    he = lax.dot_general(we_ref[...], sb, (((0,), (0,)), ((), ())),
                         preferred_element_type=jnp.float32) + be_ref[...]
    ho = lax.dot_general(wo_ref[...], sb, (((0,), (0,)), ((), ())),
                         preferred_element_type=jnp.float32) + bo_ref[...]
    he = jnp.maximum(he, 0.0) * ns
    ho = jnp.maximum(ho, 0.0) * ns
    h_ref[...] = _pack_pairs(he, ho)


_tc_mid = pl.pallas_call(
    _mid_body,
    out_shape=jax.ShapeDtypeStruct((H // 2, N), jnp.int32),
)


def _head_body(s_ref, w2_ref, b2_ref, wfc_ref, bfc_ref, nd_ref, o_ref):
    sb = s_ref[...] * nd_ref[...]
    h2 = lax.dot_general(sb, w2_ref[...], (((0,), (0,)), ((), ())),
                         preferred_element_type=jnp.float32)
    h2 = h2 + b2_ref[...]
    o_ref[...] = lax.dot_general(h2, wfc_ref[...], (((1,), (0,)), ((), ())),
                                 preferred_element_type=jnp.float32) + bfc_ref[...]


_tc_head = pl.pallas_call(
    _head_body,
    out_shape=jax.ShapeDtypeStruct((N, C), jnp.float32),
)


def kernel(x, edge_index, W1, b1, W2, b2, Wfc, bfc):
    ei = edge_index.astype(jnp.int32)
    src = ei[0]
    dst = ei[1]
    counts, pk = _sc_counts(src, dst)
    cs = counts[:, :N]
    cd = counts[:, NOFF:NOFF + N]
    p1t, ns, nd = _tc_prep(x, cs, cd)
    s1t = _sc_aggregate(p1t.reshape(-1), pk).reshape(F, N)
    h1t = _tc_mid(s1t, W1[:, 0::2], W1[:, 1::2],
                  b1[0::2].reshape(H // 2, 1), b1[1::2].reshape(H // 2, 1),
                  nd, ns)
    s2t = _sc_aggregate(h1t.reshape(-1), pk).reshape(F, N)
    out = _tc_head(s2t, W2, b2.reshape(1, H), Wfc, bfc.reshape(1, C), nd)
    return out
